# baseline (speedup 1.0000x reference)
import functools

import jax
import jax.numpy as jnp
from jax import lax
from jax.experimental import pallas as pl
from jax.experimental.pallas import tpu as pltpu
from jax.experimental.pallas import tpu_sc as plsc

NC = 2
NS = 16
LN = 16
CH = 128
NH = 2


def _ceil_to(a, m):
    return -(-a // m) * m


def _row_chunks(total, mx):
    out = []
    while total > 0:
        sz = min(mx, total)
        out.append(sz)
        total -= sz
    return out


def _deg_body(N_pad, NCH, src_hbm, dst_hbm, deg_hbm,
              idx_v, ones_v, zero_v, hist_sh):
    c = lax.axis_index("c")
    s = lax.axis_index("s")
    rpt = N_pad // NS
    base = s * rpt

    for j in range(CH // LN):
        ones_v[pl.ds(j * LN, LN)] = jnp.ones((LN,), jnp.float32)
        zero_v[pl.ds(j * LN, LN)] = jnp.zeros((LN,), jnp.float32)

    off = 0
    for sz in _row_chunks(rpt, CH):
        pltpu.sync_copy(zero_v.at[pl.ds(0, sz)], hist_sh.at[pl.ds(base + off, sz)])
        off += sz

    row_base = s * NCH

    @pl.when(c == 0)
    def _():
        pltpu.sync_copy(src_hbm.at[pl.ds(row_base, NCH)], idx_v)

    @pl.when(c == 1)
    def _():
        pltpu.sync_copy(dst_hbm.at[pl.ds(row_base, NCH)], idx_v)

    plsc.subcore_barrier()

    def chunk(j, carry):
        pltpu.sync_copy(ones_v, hist_sh.at[idx_v.at[j]], add=True)
        return carry

    lax.fori_loop(0, NCH, chunk, 0)
    plsc.subcore_barrier()

    off = 0
    for sz in _row_chunks(rpt, CH):
        pltpu.sync_copy(hist_sh.at[pl.ds(base + off, sz)], zero_v.at[pl.ds(0, sz)])
        pltpu.sync_copy(zero_v.at[pl.ds(0, sz)],
                        deg_hbm.at[c, pl.ds(base + off, sz)])
        off += sz


def _mp_body(N_pad, DH, NCH, f0_hbm, f1_hbm, src_hbm, dst_hbm, nin_hbm,
             out_hbm, idx_s, idx_d, b0, b1, b2, b3, b4, zb, nin_v,
             g0, g1, g2, g3, g4, t0, t1, t2, t3, t4, agg_sh):
    c = lax.axis_index("c")
    s = lax.axis_index("s")
    rpt = N_pad // NS
    base = s * rpt
    bufs = [b0, b1, b2, b3, b4]
    gsem = [g0, g1, g2, g3, g4]
    tsem = [t0, t1, t2, t3, t4]
    NB = 5
    LA = 3

    row_base = s * NCH
    pltpu.sync_copy(src_hbm.at[pl.ds(row_base, NCH)], idx_s)
    pltpu.sync_copy(dst_hbm.at[pl.ds(row_base, NCH)], idx_d)
    pltpu.sync_copy(nin_hbm.at[pl.ds(base, rpt)], nin_v.at[pl.ds(0, rpt)])

    def pro(feat):
        for k in range(LA):
            pltpu.async_copy(feat.at[idx_s.at[k]], bufs[k], gsem[k])

    @pl.when(c == 0)
    def _():
        pro(f0_hbm)

    @pl.when(c == 1)
    def _():
        pro(f1_hbm)

    def zrow(i, carry):
        for j in range(DH // LN):
            zb[i, pl.ds(j * LN, LN)] = jnp.zeros((LN,), jnp.float32)
        return carry

    lax.fori_loop(0, CH, zrow, 0)
    off = 0
    for sz in _row_chunks(rpt, CH):
        pltpu.sync_copy(zb.at[pl.ds(0, sz)], agg_sh.at[pl.ds(base + off, sz)])
        off += sz
    plsc.subcore_barrier()

    def pipeline(feat):
        def group(g, carry):
            for k in range(NB):
                j = g * NB + k

                @pl.when(j < NCH)
                def _():
                    kn = (k + LA) % NB

                    @pl.when(j + LA < NCH)
                    def _():
                        @pl.when(j + LA - NB >= 0)
                        def _():
                            pltpu.make_async_copy(
                                bufs[kn], agg_sh.at[idx_d.at[0]], tsem[kn]
                            ).wait()

                        pltpu.async_copy(feat.at[idx_s.at[j + LA]],
                                         bufs[kn], gsem[kn])

                    pltpu.make_async_copy(feat.at[idx_s.at[j]],
                                          bufs[k], gsem[k]).wait()
                    pltpu.async_copy(bufs[k], agg_sh.at[idx_d.at[j]],
                                     tsem[k], add=True)

            return carry

        lax.fori_loop(0, -(-NCH // NB), group, 0)
        for k in range(NB):
            pltpu.make_async_copy(bufs[k], agg_sh.at[idx_d.at[0]],
                                  tsem[k]).wait()

    @pl.when(c == 0)
    def _():
        pipeline(f0_hbm)

    @pl.when(c == 1)
    def _():
        pipeline(f1_hbm)

    plsc.subcore_barrier()

    off = 0
    for sz in _row_chunks(rpt, CH):
        pltpu.sync_copy(agg_sh.at[pl.ds(base + off, sz)], zb.at[pl.ds(0, sz)])

        def scale(r, carry):
            v = nin_v[pl.ds(off + r, LN)][0]
            for j in range(DH // LN):
                zb[r, pl.ds(j * LN, LN)] = zb[r, pl.ds(j * LN, LN)] * v
            return carry

        lax.fori_loop(0, sz, scale, 0)
        pltpu.sync_copy(zb.at[pl.ds(0, sz)],
                        out_hbm.at[c, pl.ds(base + off, sz)])
        off += sz


def _pre_body(DH, x_ref, do_col, di_row, xn0_ref, xn1_ref, nin_ref):
    n_out = lax.rsqrt(jnp.maximum(do_col[...], 1.0))
    xn = x_ref[...] * n_out
    xn0_ref[...] = xn[:, :DH]
    xn1_ref[...] = xn[:, DH:]
    nin_ref[...] = lax.rsqrt(jnp.maximum(di_row[...], 1.0))


def _layer1_body(n_valid, DH, s0, s1, w_ref, b_ref, do_col, zn0_ref, zn1_ref):
    z = (jnp.dot(s0[...], w_ref[:DH, :],
                 preferred_element_type=jnp.float32,
                 precision=lax.Precision.HIGHEST)
         + jnp.dot(s1[...], w_ref[DH:, :],
                   preferred_element_type=jnp.float32,
                   precision=lax.Precision.HIGHEST))
    z = jnp.maximum(z + b_ref[...], 0.0)
    n_out = lax.rsqrt(jnp.maximum(do_col[...], 1.0))
    blk = z.shape[0]
    row = pl.program_id(0) * blk + lax.broadcasted_iota(jnp.int32, (blk, 1), 0)
    zn = jnp.where(row < n_valid, z * n_out, 0.0)
    zn0_ref[...] = zn[:, :DH]
    zn1_ref[...] = zn[:, DH:]


def _layer2_body(DH, s0, s1, w_ref, b_ref, out_ref):
    z = (jnp.dot(s0[...], w_ref[:DH, :],
                 preferred_element_type=jnp.float32,
                 precision=lax.Precision.HIGHEST)
         + jnp.dot(s1[...], w_ref[DH:, :],
                   preferred_element_type=jnp.float32,
                   precision=lax.Precision.HIGHEST))
    out_ref[...] = z + b_ref[...]


@jax.jit
def kernel(x, edge_index, W1, b1, W2, b2):
    N, D = x.shape
    E = edge_index.shape[1]
    DH = D // NH
    N_pad = _ceil_to(N + 1, CH)
    NCH = -(-E // (NS * CH))
    E_pad = NS * NCH * CH

    pad = jnp.full((E_pad - E,), N, jnp.int32)
    src = jnp.concatenate([edge_index[0], pad]).reshape(-1, CH)
    dst = jnp.concatenate([edge_index[1], pad]).reshape(-1, CH)
    x_pad = jnp.pad(x, ((0, N_pad - N), (0, 0)))

    mesh = plsc.VectorSubcoreMesh(core_axis_name="c", subcore_axis_name="s",
                                  num_cores=NC, num_subcores=NS)

    deg_call = pl.kernel(
        functools.partial(_deg_body, N_pad, NCH),
        out_type=jax.ShapeDtypeStruct((NC, N_pad), jnp.float32),
        mesh=mesh,
        scratch_types=[
            pltpu.VMEM((NCH, CH), jnp.int32),
            pltpu.VMEM((CH,), jnp.float32),
            pltpu.VMEM((CH,), jnp.float32),
            pltpu.VMEM_SHARED((N_pad,), jnp.float32),
        ],
        compiler_params=pltpu.CompilerParams(use_tc_tiling_on_sc=False),
    )
    deg = deg_call(src, dst)
    do_col = deg[0].reshape(N_pad, 1)
    di_row = deg[1].reshape(1, N_pad)

    mp_call = pl.kernel(
        functools.partial(_mp_body, N_pad, DH, NCH),
        out_type=jax.ShapeDtypeStruct((NC, N_pad, DH), jnp.float32),
        mesh=mesh,
        scratch_types=[
            pltpu.VMEM((NCH, CH), jnp.int32),
            pltpu.VMEM((NCH, CH), jnp.int32),
            pltpu.VMEM((CH, DH), jnp.float32),
            pltpu.VMEM((CH, DH), jnp.float32),
            pltpu.VMEM((CH, DH), jnp.float32),
            pltpu.VMEM((CH, DH), jnp.float32),
            pltpu.VMEM((CH, DH), jnp.float32),
            pltpu.VMEM((CH, DH), jnp.float32),
            pltpu.VMEM((N_pad // NS + LN,), jnp.float32),
            pltpu.SemaphoreType.DMA,
            pltpu.SemaphoreType.DMA,
            pltpu.SemaphoreType.DMA,
            pltpu.SemaphoreType.DMA,
            pltpu.SemaphoreType.DMA,
            pltpu.SemaphoreType.DMA,
            pltpu.SemaphoreType.DMA,
            pltpu.SemaphoreType.DMA,
            pltpu.SemaphoreType.DMA,
            pltpu.SemaphoreType.DMA,
            pltpu.VMEM_SHARED((N_pad, DH), jnp.float32),
        ],
        compiler_params=pltpu.CompilerParams(use_tc_tiling_on_sc=False),
    )

    tc_grid = 2
    blk = N_pad // tc_grid
    col_spec = pl.BlockSpec((blk, 1), lambda i: (i, 0))
    mat_spec = pl.BlockSpec((blk, D), lambda i: (i, 0))
    half_spec = pl.BlockSpec((blk, DH), lambda i: (i, 0))
    w_spec = pl.BlockSpec((D, D), lambda i: (0, 0))
    b_spec = pl.BlockSpec((1, D), lambda i: (0, 0))
    row_spec = pl.BlockSpec((1, N_pad), lambda i: (0, 0))

    xn0, xn1, nin2d = pl.pallas_call(
        functools.partial(_pre_body, DH),
        grid=(tc_grid,),
        in_specs=[mat_spec, col_spec, row_spec],
        out_specs=[half_spec, half_spec, row_spec],
        out_shape=[jax.ShapeDtypeStruct((N_pad, DH), jnp.float32),
                   jax.ShapeDtypeStruct((N_pad, DH), jnp.float32),
                   jax.ShapeDtypeStruct((1, N_pad), jnp.float32)],
    )(x_pad, do_col, di_row)
    nin = nin2d.reshape(N_pad)

    S1 = mp_call(xn0, xn1, src, dst, nin)

    zn0, zn1 = pl.pallas_call(
        functools.partial(_layer1_body, N, DH),
        grid=(tc_grid,),
        in_specs=[half_spec, half_spec, w_spec, b_spec, col_spec],
        out_specs=[half_spec, half_spec],
        out_shape=[jax.ShapeDtypeStruct((N_pad, DH), jnp.float32),
                   jax.ShapeDtypeStruct((N_pad, DH), jnp.float32)],
    )(S1[0], S1[1], W1, b1.reshape(1, D), do_col)

    S2 = mp_call(zn0, zn1, src, dst, nin)

    out = pl.pallas_call(
        functools.partial(_layer2_body, DH),
        grid=(tc_grid,),
        in_specs=[half_spec, half_spec, w_spec, b_spec],
        out_specs=mat_spec,
        out_shape=jax.ShapeDtypeStruct((N_pad, D), jnp.float32),
    )(S2[0], S2[1], W2, b2.reshape(1, D))

    return out[:N]


# speedup vs baseline: 1.0252x; 1.0252x over previous
import functools

import jax
import jax.numpy as jnp
from jax import lax
from jax.experimental import pallas as pl
from jax.experimental.pallas import tpu as pltpu
from jax.experimental.pallas import tpu_sc as plsc

NC = 2
NS = 16
LN = 16
CH = 128
NH = 2


def _ceil_to(a, m):
    return -(-a // m) * m


def _row_chunks(total, mx):
    out = []
    while total > 0:
        sz = min(mx, total)
        out.append(sz)
        total -= sz
    return out


def _deg_body(N_pad, NCH, src_hbm, dst_hbm, deg_hbm,
              idx_v, ones_v, zero_v, hist_sh):
    c = lax.axis_index("c")
    s = lax.axis_index("s")
    rpt = N_pad // NS
    base = s * rpt

    for j in range(CH // LN):
        ones_v[pl.ds(j * LN, LN)] = jnp.ones((LN,), jnp.float32)
        zero_v[pl.ds(j * LN, LN)] = jnp.zeros((LN,), jnp.float32)

    off = 0
    for sz in _row_chunks(rpt, CH):
        pltpu.sync_copy(zero_v.at[pl.ds(0, sz)], hist_sh.at[pl.ds(base + off, sz)])
        off += sz

    row_base = s * NCH

    @pl.when(c == 0)
    def _():
        pltpu.sync_copy(src_hbm.at[pl.ds(row_base, NCH)], idx_v)

    @pl.when(c == 1)
    def _():
        pltpu.sync_copy(dst_hbm.at[pl.ds(row_base, NCH)], idx_v)

    plsc.subcore_barrier()

    def chunk(j, carry):
        pltpu.sync_copy(ones_v, hist_sh.at[idx_v.at[j]], add=True)
        return carry

    lax.fori_loop(0, NCH, chunk, 0)
    plsc.subcore_barrier()

    off = 0
    for sz in _row_chunks(rpt, CH):
        pltpu.sync_copy(hist_sh.at[pl.ds(base + off, sz)], zero_v.at[pl.ds(0, sz)])
        pltpu.sync_copy(zero_v.at[pl.ds(0, sz)],
                        deg_hbm.at[c, pl.ds(base + off, sz)])
        off += sz


def _mp_body(N_pad, DH, NCH, f0_hbm, f1_hbm, src_hbm, dst_hbm, nin_hbm,
             out_hbm, idx_s, idx_d, b0, b1, b2, b3, b4, zb, nin_v,
             g0, g1, g2, g3, g4, t0, t1, t2, t3, t4, agg_sh):
    c = lax.axis_index("c")
    s = lax.axis_index("s")
    rpt = N_pad // NS
    base = s * rpt
    bufs = [b0, b1, b2, b3, b4]
    gsem = [g0, g1, g2, g3, g4]
    tsem = [t0, t1, t2, t3, t4]
    NB = 5
    LA = 3

    row_base = s * NCH
    pltpu.sync_copy(src_hbm.at[pl.ds(row_base, NCH)], idx_s)
    pltpu.sync_copy(dst_hbm.at[pl.ds(row_base, NCH)], idx_d)
    pltpu.sync_copy(nin_hbm.at[pl.ds(base, rpt)], nin_v.at[pl.ds(0, rpt)])

    def pro(feat):
        for k in range(LA):
            pltpu.async_copy(feat.at[idx_s.at[k]], bufs[k], gsem[k])

    @pl.when(c == 0)
    def _():
        pro(f0_hbm)

    @pl.when(c == 1)
    def _():
        pro(f1_hbm)

    def zrow(i, carry):
        for j in range(DH // LN):
            zb[i, pl.ds(j * LN, LN)] = jnp.zeros((LN,), jnp.float32)
        return carry

    lax.fori_loop(0, CH, zrow, 0)
    off = 0
    for sz in _row_chunks(rpt, CH):
        pltpu.sync_copy(zb.at[pl.ds(0, sz)], agg_sh.at[pl.ds(base + off, sz)])
        off += sz
    plsc.subcore_barrier()

    def pipeline(feat):
        def group(g, carry):
            for k in range(NB):
                j = g * NB + k

                @pl.when(j < NCH)
                def _():
                    kn = (k + LA) % NB

                    @pl.when(j + LA < NCH)
                    def _():
                        @pl.when(j + LA - NB >= 0)
                        def _():
                            pltpu.make_async_copy(
                                bufs[kn], agg_sh.at[idx_d.at[0]], tsem[kn]
                            ).wait()

                        pltpu.async_copy(feat.at[idx_s.at[j + LA]],
                                         bufs[kn], gsem[kn])

                    pltpu.make_async_copy(feat.at[idx_s.at[j]],
                                          bufs[k], gsem[k]).wait()
                    pltpu.async_copy(bufs[k], agg_sh.at[idx_d.at[j]],
                                     tsem[k], add=True)

            return carry

        lax.fori_loop(0, -(-NCH // NB), group, 0)
        for k in range(NB):
            pltpu.make_async_copy(bufs[k], agg_sh.at[idx_d.at[0]],
                                  tsem[k]).wait()

    @pl.when(c == 0)
    def _():
        pipeline(f0_hbm)

    @pl.when(c == 1)
    def _():
        pipeline(f1_hbm)

    plsc.subcore_barrier()

    off = 0
    for sz in _row_chunks(rpt, CH):
        pltpu.sync_copy(agg_sh.at[pl.ds(base + off, sz)], zb.at[pl.ds(0, sz)])

        def scale(r, carry):
            v = nin_v[pl.ds(off + r, LN)][0]
            for j in range(DH // LN):
                zb[r, pl.ds(j * LN, LN)] = zb[r, pl.ds(j * LN, LN)] * v
            return carry

        lax.fori_loop(0, sz, scale, 0)
        pltpu.sync_copy(zb.at[pl.ds(0, sz)],
                        out_hbm.at[c, pl.ds(base + off, sz)])
        off += sz


def _pre_body(DH, x_ref, do_col, di_row, xn0_ref, xn1_ref, nin_ref):
    n_out = lax.rsqrt(jnp.maximum(do_col[...], 1.0))
    xn = x_ref[...] * n_out
    xn0_ref[...] = xn[:, :DH]
    xn1_ref[...] = xn[:, DH:]
    nin_ref[...] = lax.rsqrt(jnp.maximum(di_row[...], 1.0))


def _layer1_body(n_valid, DH, s0, s1, w_ref, b_ref, do_col, zn0_ref, zn1_ref):
    z = (jnp.dot(s0[...], w_ref[:DH, :],
                 preferred_element_type=jnp.float32,
                 precision=lax.Precision.HIGHEST)
         + jnp.dot(s1[...], w_ref[DH:, :],
                   preferred_element_type=jnp.float32,
                   precision=lax.Precision.HIGHEST))
    z = jnp.maximum(z + b_ref[...], 0.0)
    n_out = lax.rsqrt(jnp.maximum(do_col[...], 1.0))
    blk = z.shape[0]
    row = pl.program_id(0) * blk + lax.broadcasted_iota(jnp.int32, (blk, 1), 0)
    zn = jnp.where(row < n_valid, z * n_out, 0.0)
    zn0_ref[...] = zn[:, :DH]
    zn1_ref[...] = zn[:, DH:]


def _layer2_body(DH, s0, s1, w_ref, b_ref, out_ref):
    z = (jnp.dot(s0[...], w_ref[:DH, :],
                 preferred_element_type=jnp.float32,
                 precision=lax.Precision.HIGHEST)
         + jnp.dot(s1[...], w_ref[DH:, :],
                   preferred_element_type=jnp.float32,
                   precision=lax.Precision.HIGHEST))
    out_ref[...] = z + b_ref[...]


@jax.jit
def kernel(x, edge_index, W1, b1, W2, b2):
    N, D = x.shape
    E = edge_index.shape[1]
    DH = D // NH
    N_pad = _ceil_to(N + 1, CH)
    NCH = -(-E // (NS * CH))
    E_pad = NS * NCH * CH

    ep = jnp.pad(edge_index, ((0, 0), (0, E_pad - E)), constant_values=N)
    src = ep[0].reshape(-1, CH)
    dst = ep[1].reshape(-1, CH)
    x_pad = jnp.pad(x, ((0, N_pad - N), (0, 0)))

    mesh = plsc.VectorSubcoreMesh(core_axis_name="c", subcore_axis_name="s",
                                  num_cores=NC, num_subcores=NS)

    deg_call = pl.kernel(
        functools.partial(_deg_body, N_pad, NCH),
        out_type=jax.ShapeDtypeStruct((NC, N_pad), jnp.float32),
        mesh=mesh,
        scratch_types=[
            pltpu.VMEM((NCH, CH), jnp.int32),
            pltpu.VMEM((CH,), jnp.float32),
            pltpu.VMEM((CH,), jnp.float32),
            pltpu.VMEM_SHARED((N_pad,), jnp.float32),
        ],
        compiler_params=pltpu.CompilerParams(use_tc_tiling_on_sc=False),
    )
    deg = deg_call(src, dst)
    do_col = deg[0].reshape(N_pad, 1)
    di_row = deg[1].reshape(1, N_pad)

    mp_call = pl.kernel(
        functools.partial(_mp_body, N_pad, DH, NCH),
        out_type=jax.ShapeDtypeStruct((NC, N_pad, DH), jnp.float32),
        mesh=mesh,
        scratch_types=[
            pltpu.VMEM((NCH, CH), jnp.int32),
            pltpu.VMEM((NCH, CH), jnp.int32),
            pltpu.VMEM((CH, DH), jnp.float32),
            pltpu.VMEM((CH, DH), jnp.float32),
            pltpu.VMEM((CH, DH), jnp.float32),
            pltpu.VMEM((CH, DH), jnp.float32),
            pltpu.VMEM((CH, DH), jnp.float32),
            pltpu.VMEM((CH, DH), jnp.float32),
            pltpu.VMEM((N_pad // NS + LN,), jnp.float32),
            pltpu.SemaphoreType.DMA,
            pltpu.SemaphoreType.DMA,
            pltpu.SemaphoreType.DMA,
            pltpu.SemaphoreType.DMA,
            pltpu.SemaphoreType.DMA,
            pltpu.SemaphoreType.DMA,
            pltpu.SemaphoreType.DMA,
            pltpu.SemaphoreType.DMA,
            pltpu.SemaphoreType.DMA,
            pltpu.SemaphoreType.DMA,
            pltpu.VMEM_SHARED((N_pad, DH), jnp.float32),
        ],
        compiler_params=pltpu.CompilerParams(use_tc_tiling_on_sc=False),
    )

    tc_grid = 4
    blk = N_pad // tc_grid
    col_spec = pl.BlockSpec((blk, 1), lambda i: (i, 0))
    mat_spec = pl.BlockSpec((blk, D), lambda i: (i, 0))
    half_spec = pl.BlockSpec((blk, DH), lambda i: (i, 0))
    w_spec = pl.BlockSpec((D, D), lambda i: (0, 0))
    b_spec = pl.BlockSpec((1, D), lambda i: (0, 0))
    row_spec = pl.BlockSpec((1, N_pad), lambda i: (0, 0))

    xn0, xn1, nin2d = pl.pallas_call(
        functools.partial(_pre_body, DH),
        grid=(tc_grid,),
        in_specs=[mat_spec, col_spec, row_spec],
        out_specs=[half_spec, half_spec, row_spec],
        out_shape=[jax.ShapeDtypeStruct((N_pad, DH), jnp.float32),
                   jax.ShapeDtypeStruct((N_pad, DH), jnp.float32),
                   jax.ShapeDtypeStruct((1, N_pad), jnp.float32)],
    )(x_pad, do_col, di_row)
    nin = nin2d.reshape(N_pad)

    S1 = mp_call(xn0, xn1, src, dst, nin)

    zn0, zn1 = pl.pallas_call(
        functools.partial(_layer1_body, N, DH),
        grid=(tc_grid,),
        in_specs=[half_spec, half_spec, w_spec, b_spec, col_spec],
        out_specs=[half_spec, half_spec],
        out_shape=[jax.ShapeDtypeStruct((N_pad, DH), jnp.float32),
                   jax.ShapeDtypeStruct((N_pad, DH), jnp.float32)],
    )(S1[0], S1[1], W1, b1.reshape(1, D), do_col)

    S2 = mp_call(zn0, zn1, src, dst, nin)

    out = pl.pallas_call(
        functools.partial(_layer2_body, DH),
        grid=(tc_grid,),
        in_specs=[half_spec, half_spec, w_spec, b_spec],
        out_specs=mat_spec,
        out_shape=jax.ShapeDtypeStruct((N_pad, D), jnp.float32),
    )(S2[0], S2[1], W2, b2.reshape(1, D))

    return out[:N]


# speedup vs baseline: 1.0476x; 1.0218x over previous
import functools

import jax
import jax.numpy as jnp
from jax import lax
from jax.experimental import pallas as pl
from jax.experimental.pallas import tpu as pltpu
from jax.experimental.pallas import tpu_sc as plsc

NC = 2
NS = 16
LN = 16
CH = 128
NH = 2


def _ceil_to(a, m):
    return -(-a // m) * m


def _row_chunks(total, mx):
    out = []
    while total > 0:
        sz = min(mx, total)
        out.append(sz)
        total -= sz
    return out


def _deg_body(N_pad, NCH, src_hbm, dst_hbm, deg_hbm,
              idx_v, ones_v, zero_v, hist_sh):
    c = lax.axis_index("c")
    s = lax.axis_index("s")
    rpt = N_pad // NS
    base = s * rpt

    for j in range(CH // LN):
        ones_v[pl.ds(j * LN, LN)] = jnp.ones((LN,), jnp.float32)
        zero_v[pl.ds(j * LN, LN)] = jnp.zeros((LN,), jnp.float32)

    off = 0
    for sz in _row_chunks(rpt, CH):
        pltpu.sync_copy(zero_v.at[pl.ds(0, sz)], hist_sh.at[pl.ds(base + off, sz)])
        off += sz

    row_base = s * NCH

    @pl.when(c == 0)
    def _():
        pltpu.sync_copy(src_hbm.at[pl.ds(row_base, NCH)], idx_v)

    @pl.when(c == 1)
    def _():
        pltpu.sync_copy(dst_hbm.at[pl.ds(row_base, NCH)], idx_v)

    plsc.subcore_barrier()

    def chunk(j, carry):
        pltpu.sync_copy(ones_v, hist_sh.at[idx_v.at[j]], add=True)
        return carry

    lax.fori_loop(0, NCH, chunk, 0)
    plsc.subcore_barrier()

    off = 0
    for sz in _row_chunks(rpt, CH):
        pltpu.sync_copy(hist_sh.at[pl.ds(base + off, sz)], zero_v.at[pl.ds(0, sz)])
        pltpu.sync_copy(zero_v.at[pl.ds(0, sz)],
                        deg_hbm.at[c, pl.ds(base + off, sz)])
        off += sz


def _mp_body(N_pad, DH, NCH, f0_hbm, f1_hbm, src_hbm, dst_hbm, nin_hbm,
             out_hbm, idx_s, idx_d, b0, b1, b2, b3, b4, zb, nin_v,
             g0, g1, g2, g3, g4, t0, t1, t2, t3, t4, agg_sh):
    c = lax.axis_index("c")
    s = lax.axis_index("s")
    rpt = N_pad // NS
    base = s * rpt
    bufs = [b0, b1, b2, b3, b4]
    gsem = [g0, g1, g2, g3, g4]
    tsem = [t0, t1, t2, t3, t4]
    NB = 5
    LA = 4

    row_base = s * NCH
    pltpu.sync_copy(src_hbm.at[pl.ds(row_base, NCH)], idx_s)
    pltpu.sync_copy(dst_hbm.at[pl.ds(row_base, NCH)], idx_d)
    pltpu.sync_copy(nin_hbm.at[pl.ds(base, rpt)], nin_v.at[pl.ds(0, rpt)])

    def pro(feat):
        for k in range(LA):
            pltpu.async_copy(feat.at[idx_s.at[k]], bufs[k], gsem[k])

    @pl.when(c == 0)
    def _():
        pro(f0_hbm)

    @pl.when(c == 1)
    def _():
        pro(f1_hbm)

    def zrow(i, carry):
        for j in range(DH // LN):
            zb[i, pl.ds(j * LN, LN)] = jnp.zeros((LN,), jnp.float32)
        return carry

    lax.fori_loop(0, CH, zrow, 0)
    off = 0
    for sz in _row_chunks(rpt, CH):
        pltpu.sync_copy(zb.at[pl.ds(0, sz)], agg_sh.at[pl.ds(base + off, sz)])
        off += sz
    plsc.subcore_barrier()

    def pipeline(feat):
        def group(g, carry):
            for k in range(NB):
                j = g * NB + k

                @pl.when(j < NCH)
                def _():
                    kn = (k + LA) % NB

                    @pl.when(j + LA < NCH)
                    def _():
                        @pl.when(j + LA - NB >= 0)
                        def _():
                            pltpu.make_async_copy(
                                bufs[kn], agg_sh.at[idx_d.at[0]], tsem[kn]
                            ).wait()

                        pltpu.async_copy(feat.at[idx_s.at[j + LA]],
                                         bufs[kn], gsem[kn])

                    pltpu.make_async_copy(feat.at[idx_s.at[j]],
                                          bufs[k], gsem[k]).wait()
                    pltpu.async_copy(bufs[k], agg_sh.at[idx_d.at[j]],
                                     tsem[k], add=True)

            return carry

        lax.fori_loop(0, -(-NCH // NB), group, 0)
        for k in range(NB):
            pltpu.make_async_copy(bufs[k], agg_sh.at[idx_d.at[0]],
                                  tsem[k]).wait()

    @pl.when(c == 0)
    def _():
        pipeline(f0_hbm)

    @pl.when(c == 1)
    def _():
        pipeline(f1_hbm)

    plsc.subcore_barrier()

    off = 0
    for sz in _row_chunks(rpt, CH):
        pltpu.sync_copy(agg_sh.at[pl.ds(base + off, sz)], zb.at[pl.ds(0, sz)])

        def scale(r, carry):
            v = nin_v[pl.ds(off + r, LN)][0]
            for j in range(DH // LN):
                zb[r, pl.ds(j * LN, LN)] = zb[r, pl.ds(j * LN, LN)] * v
            return carry

        lax.fori_loop(0, sz, scale, 0)
        pltpu.sync_copy(zb.at[pl.ds(0, sz)],
                        out_hbm.at[c, pl.ds(base + off, sz)])
        off += sz


def _pre_body(DH, x_ref, do_col, di_row, xn0_ref, xn1_ref, nin_ref):
    n_out = lax.rsqrt(jnp.maximum(do_col[...], 1.0))
    xn = x_ref[...] * n_out
    xn0_ref[...] = xn[:, :DH]
    xn1_ref[...] = xn[:, DH:]
    nin_ref[...] = lax.rsqrt(jnp.maximum(di_row[...], 1.0))


def _layer1_body(n_valid, DH, s0, s1, w_ref, b_ref, do_col, zn0_ref, zn1_ref):
    z = (jnp.dot(s0[...], w_ref[:DH, :],
                 preferred_element_type=jnp.float32,
                 precision=lax.Precision.HIGHEST)
         + jnp.dot(s1[...], w_ref[DH:, :],
                   preferred_element_type=jnp.float32,
                   precision=lax.Precision.HIGHEST))
    z = jnp.maximum(z + b_ref[...], 0.0)
    n_out = lax.rsqrt(jnp.maximum(do_col[...], 1.0))
    blk = z.shape[0]
    row = pl.program_id(0) * blk + lax.broadcasted_iota(jnp.int32, (blk, 1), 0)
    zn = jnp.where(row < n_valid, z * n_out, 0.0)
    zn0_ref[...] = zn[:, :DH]
    zn1_ref[...] = zn[:, DH:]


def _layer2_body(DH, s0, s1, w_ref, b_ref, out_ref):
    z = (jnp.dot(s0[...], w_ref[:DH, :],
                 preferred_element_type=jnp.float32,
                 precision=lax.Precision.HIGHEST)
         + jnp.dot(s1[...], w_ref[DH:, :],
                   preferred_element_type=jnp.float32,
                   precision=lax.Precision.HIGHEST))
    out_ref[...] = z + b_ref[...]


@jax.jit
def kernel(x, edge_index, W1, b1, W2, b2):
    N, D = x.shape
    E = edge_index.shape[1]
    DH = D // NH
    N_pad = _ceil_to(N + 1, CH)
    NCH = -(-E // (NS * CH))
    E_pad = NS * NCH * CH

    ep = jnp.pad(edge_index, ((0, 0), (0, E_pad - E)), constant_values=N)
    src = ep[0].reshape(-1, CH)
    dst = ep[1].reshape(-1, CH)
    x_pad = jnp.pad(x, ((0, N_pad - N), (0, 0)))

    mesh = plsc.VectorSubcoreMesh(core_axis_name="c", subcore_axis_name="s",
                                  num_cores=NC, num_subcores=NS)

    deg_call = pl.kernel(
        functools.partial(_deg_body, N_pad, NCH),
        out_type=jax.ShapeDtypeStruct((NC, N_pad), jnp.float32),
        mesh=mesh,
        scratch_types=[
            pltpu.VMEM((NCH, CH), jnp.int32),
            pltpu.VMEM((CH,), jnp.float32),
            pltpu.VMEM((CH,), jnp.float32),
            pltpu.VMEM_SHARED((N_pad,), jnp.float32),
        ],
        compiler_params=pltpu.CompilerParams(use_tc_tiling_on_sc=False),
    )
    deg = deg_call(src, dst)
    do_col = deg[0].reshape(N_pad, 1)
    di_row = deg[1].reshape(1, N_pad)

    mp_call = pl.kernel(
        functools.partial(_mp_body, N_pad, DH, NCH),
        out_type=jax.ShapeDtypeStruct((NC, N_pad, DH), jnp.float32),
        mesh=mesh,
        scratch_types=[
            pltpu.VMEM((NCH, CH), jnp.int32),
            pltpu.VMEM((NCH, CH), jnp.int32),
            pltpu.VMEM((CH, DH), jnp.float32),
            pltpu.VMEM((CH, DH), jnp.float32),
            pltpu.VMEM((CH, DH), jnp.float32),
            pltpu.VMEM((CH, DH), jnp.float32),
            pltpu.VMEM((CH, DH), jnp.float32),
            pltpu.VMEM((CH, DH), jnp.float32),
            pltpu.VMEM((N_pad // NS + LN,), jnp.float32),
            pltpu.SemaphoreType.DMA,
            pltpu.SemaphoreType.DMA,
            pltpu.SemaphoreType.DMA,
            pltpu.SemaphoreType.DMA,
            pltpu.SemaphoreType.DMA,
            pltpu.SemaphoreType.DMA,
            pltpu.SemaphoreType.DMA,
            pltpu.SemaphoreType.DMA,
            pltpu.SemaphoreType.DMA,
            pltpu.SemaphoreType.DMA,
            pltpu.VMEM_SHARED((N_pad, DH), jnp.float32),
        ],
        compiler_params=pltpu.CompilerParams(use_tc_tiling_on_sc=False),
    )

    tc_grid = 4
    blk = N_pad // tc_grid
    col_spec = pl.BlockSpec((blk, 1), lambda i: (i, 0))
    mat_spec = pl.BlockSpec((blk, D), lambda i: (i, 0))
    half_spec = pl.BlockSpec((blk, DH), lambda i: (i, 0))
    w_spec = pl.BlockSpec((D, D), lambda i: (0, 0))
    b_spec = pl.BlockSpec((1, D), lambda i: (0, 0))
    row_spec = pl.BlockSpec((1, N_pad), lambda i: (0, 0))

    xn0, xn1, nin2d = pl.pallas_call(
        functools.partial(_pre_body, DH),
        grid=(tc_grid,),
        in_specs=[mat_spec, col_spec, row_spec],
        out_specs=[half_spec, half_spec, row_spec],
        out_shape=[jax.ShapeDtypeStruct((N_pad, DH), jnp.float32),
                   jax.ShapeDtypeStruct((N_pad, DH), jnp.float32),
                   jax.ShapeDtypeStruct((1, N_pad), jnp.float32)],
    )(x_pad, do_col, di_row)
    nin = nin2d.reshape(N_pad)

    S1 = mp_call(xn0, xn1, src, dst, nin)

    zn0, zn1 = pl.pallas_call(
        functools.partial(_layer1_body, N, DH),
        grid=(tc_grid,),
        in_specs=[half_spec, half_spec, w_spec, b_spec, col_spec],
        out_specs=[half_spec, half_spec],
        out_shape=[jax.ShapeDtypeStruct((N_pad, DH), jnp.float32),
                   jax.ShapeDtypeStruct((N_pad, DH), jnp.float32)],
    )(S1[0], S1[1], W1, b1.reshape(1, D), do_col)

    S2 = mp_call(zn0, zn1, src, dst, nin)

    out = pl.pallas_call(
        functools.partial(_layer2_body, DH),
        grid=(tc_grid,),
        in_specs=[half_spec, half_spec, w_spec, b_spec],
        out_specs=mat_spec,
        out_shape=jax.ShapeDtypeStruct((N_pad, D), jnp.float32),
    )(S2[0], S2[1], W2, b2.reshape(1, D))

    return out[:N]
